# baseline (speedup 1.0000x reference)
import functools

import jax
import jax.numpy as jnp
from jax import lax
from jax.experimental import pallas as pl
from jax.experimental.pallas import tpu as pltpu
from jax.experimental.pallas import tpu_sc as plsc


def _mesh():
    return plsc.VectorSubcoreMesh(core_axis_name="c", subcore_axis_name="s")


def _transpose_table_kernel(num_cores):
    n_main = 244

    @functools.partial(
        pl.kernel,
        mesh=_mesh(),
        out_type=jax.ShapeDtypeStruct((64000000,), jnp.float32),
        scratch_types=[
            pltpu.VMEM((2, 64, 128), jnp.float32),
            pltpu.VMEM((2, 8192), jnp.float32),
            pltpu.SemaphoreType.DMA,
            pltpu.SemaphoreType.DMA,
        ],
        compiler_params=pltpu.CompilerParams(use_tc_tiling_on_sc=True, needs_layout_passes=False),
    )
    def k1(tt_hbm, tail_hbm, t2_hbm, colbuf, rowbuf, rsem, wsem):
        wid = lax.axis_index("s") * num_cores + lax.axis_index("c")
        iota = lax.broadcasted_iota(jnp.int32, (16,), 0)
        dvecs = [iota + 16 * h for h in range(4)]

        def read_block(b, buf):
            pltpu.async_copy(
                tt_hbm.at[:, pl.ds(128 * b, 128)], colbuf.at[buf], rsem
            )

        def wait_words(sem, nwords):
            pltpu.make_async_copy(
                t2_hbm.at[pl.ds(0, nwords)],
                rowbuf.at[0, pl.ds(0, nwords)],
                sem,
            ).wait()

        def transpose_block(buf, ncols):
            for j in range(ncols):
                js = jnp.full((16,), j, jnp.int32)
                for h in range(4):
                    vals = plsc.load_gather(colbuf.at[buf], [dvecs[h], js])
                    rowbuf[buf, pl.ds(j * 64 + 16 * h, 16)] = vals

        read_block(wid, 0)

        def outer(t, carry):
            for half in range(2):
                kk = 2 * t + half
                b = wid + 32 * kk

                @pl.when(kk + 1 < n_main)
                def _():
                    read_block(wid + 32 * (kk + 1), 1 - half)

                wait_words(rsem, 8192)

                @pl.when(kk >= 2)
                def _():
                    wait_words(wsem, 8192)

                transpose_block(half, 128)
                pltpu.async_copy(
                    rowbuf.at[half], t2_hbm.at[pl.ds(8192 * b, 8192)], wsem
                )
            return carry

        lax.fori_loop(0, n_main // 2, outer, 0)
        wait_words(wsem, 8192)
        wait_words(wsem, 8192)

        @pl.when(wid < 4)
        def _():
            b = 7808 + wid
            pltpu.sync_copy(tt_hbm.at[:, pl.ds(128 * b, 128)], colbuf.at[0])
            transpose_block(0, 128)
            pltpu.sync_copy(rowbuf.at[0], t2_hbm.at[pl.ds(8192 * b, 8192)])

        @pl.when(wid == 4)
        def _():
            pltpu.sync_copy(tail_hbm, rowbuf.at[0, pl.ds(0, 4096)])
            pltpu.sync_copy(
                rowbuf.at[0, pl.ds(0, 4096)],
                t2_hbm.at[pl.ds(64 * 999936, 4096)],
            )

    return k1


def _gather_kernel(num_cores):
    n_tasks = 200

    @functools.partial(
        pl.kernel,
        mesh=_mesh(),
        out_type=jax.ShapeDtypeStruct((200, 8, 32, 8, 128), jnp.float32),
        scratch_types=[
            pltpu.VMEM((200, 128), jnp.int32),
            pltpu.VMEM((2, 128, 64), jnp.float32),
            pltpu.VMEM((2, 8, 8, 128), jnp.float32),
            pltpu.SemaphoreType.DMA,
            pltpu.SemaphoreType.DMA,
        ],
        compiler_params=pltpu.CompilerParams(use_tc_tiling_on_sc=False, needs_layout_passes=False),
    )
    def k2(t2_hbm, xq_hbm, out_hbm, idxv, rows, outv, gsem, wsem):
        bt = lax.axis_index("s") * num_cores + lax.axis_index("c")
        iota = lax.broadcasted_iota(jnp.int32, (16,), 0)
        jvecs = [iota + 16 * g for g in range(8)]

        for st in range(25):
            pltpu.async_copy(
                xq_hbm.at[st, bt], idxv.at[pl.ds(8 * st, 8)], gsem
            )
        for _ in range(25):
            pltpu.make_async_copy(
                xq_hbm.at[0, 0], idxv.at[pl.ds(0, 8)], gsem
            ).wait()

        def start_gather(k, buf):
            pltpu.async_copy(t2_hbm.at[idxv.at[k]], rows.at[buf], gsem)

        def wait_gather():
            pltpu.make_async_copy(
                t2_hbm.at[pl.ds(0, 128)], rows.at[0], gsem
            ).wait()

        def wait_write():
            pltpu.make_async_copy(
                out_hbm.at[0, :, 0], outv.at[0], wsem
            ).wait()

        start_gather(0, 0)

        def task(k, buf):
            @pl.when(k + 1 < n_tasks)
            def _():
                start_gather(k + 1, 1 - buf)

            wait_gather()

            @pl.when(k >= 2)
            def _():
                wait_write()

            for dt in range(8):
                for ds in range(8):
                    d = 8 * dt + ds
                    dsplat = jnp.full((16,), d, jnp.int32)
                    for g in range(8):
                        vals = plsc.load_gather(
                            rows.at[buf], [jvecs[g], dsplat]
                        )
                        outv[buf, dt, ds, pl.ds(16 * g, 16)] = vals
            pltpu.async_copy(outv.at[buf], out_hbm.at[k, :, bt], wsem)
            return None

        def outer(t, carry):
            task(2 * t, 0)
            task(2 * t + 1, 1)
            return carry

        lax.fori_loop(0, n_tasks // 2, outer, 0)
        wait_write()
        wait_write()

    return k2


def kernel(x, mask, table):
    del mask
    xq = x.T.reshape(25, 8, 32, 128).transpose(0, 2, 1, 3)
    tt = table.T
    info = plsc.get_sparse_core_info()
    tail = table[999936:, :].reshape(-1)
    t2 = _transpose_table_kernel(info.num_cores)(tt, tail).reshape(1000000, 64)
    out5 = _gather_kernel(info.num_cores)(t2, xq)
    return out5.transpose(2, 4, 0, 1, 3).reshape(4096, 200, 64)


# speedup vs baseline: 1.9305x; 1.9305x over previous
import functools

import jax
import jax.numpy as jnp
from jax import lax
from jax.experimental import pallas as pl
from jax.experimental.pallas import tpu as pltpu
from jax.experimental.pallas import tpu_sc as plsc


def _mesh():
    return plsc.VectorSubcoreMesh(core_axis_name="c", subcore_axis_name="s")


def _transpose_table_kernel(num_cores):
    n_main = 244

    @functools.partial(
        pl.kernel,
        mesh=_mesh(),
        out_type=jax.ShapeDtypeStruct((64000000,), jnp.float32),
        scratch_types=[
            pltpu.VMEM((2, 64, 128), jnp.float32),
            pltpu.VMEM((2, 8192), jnp.float32),
            pltpu.SemaphoreType.DMA,
            pltpu.SemaphoreType.DMA,
        ],
        compiler_params=pltpu.CompilerParams(use_tc_tiling_on_sc=True, needs_layout_passes=False),
    )
    def k1(tt_hbm, tail_hbm, t2_hbm, colbuf, rowbuf, rsem, wsem):
        wid = lax.axis_index("s") * num_cores + lax.axis_index("c")
        iota = lax.broadcasted_iota(jnp.int32, (16,), 0)
        dvecs = [iota + 16 * h for h in range(4)]

        def read_block(b, buf):
            pltpu.async_copy(
                tt_hbm.at[:, pl.ds(128 * b, 128)], colbuf.at[buf], rsem
            )

        def wait_words(sem, nwords):
            pltpu.make_async_copy(
                t2_hbm.at[pl.ds(0, nwords)],
                rowbuf.at[0, pl.ds(0, nwords)],
                sem,
            ).wait()

        def transpose_block(buf, ncols):
            @plsc.parallel_loop(0, ncols, unroll=8)
            def _(j):
                js = jnp.full((16,), j, jnp.int32)
                for h in range(4):
                    vals = plsc.load_gather(colbuf.at[buf], [dvecs[h], js])
                    rowbuf[buf, pl.ds(j * 64 + 16 * h, 16)] = vals

        read_block(wid, 0)

        def outer(t, carry):
            for half in range(2):
                kk = 2 * t + half
                b = wid + 32 * kk

                @pl.when(kk + 1 < n_main)
                def _():
                    read_block(wid + 32 * (kk + 1), 1 - half)

                wait_words(rsem, 8192)

                @pl.when(kk >= 2)
                def _():
                    wait_words(wsem, 8192)

                transpose_block(half, 128)
                pltpu.async_copy(
                    rowbuf.at[half], t2_hbm.at[pl.ds(8192 * b, 8192)], wsem
                )
            return carry

        lax.fori_loop(0, n_main // 2, outer, 0)
        wait_words(wsem, 8192)
        wait_words(wsem, 8192)

        @pl.when(wid < 4)
        def _():
            b = 7808 + wid
            pltpu.sync_copy(tt_hbm.at[:, pl.ds(128 * b, 128)], colbuf.at[0])
            transpose_block(0, 128)
            pltpu.sync_copy(rowbuf.at[0], t2_hbm.at[pl.ds(8192 * b, 8192)])

        @pl.when(wid == 4)
        def _():
            pltpu.sync_copy(tail_hbm, rowbuf.at[0, pl.ds(0, 4096)])
            pltpu.sync_copy(
                rowbuf.at[0, pl.ds(0, 4096)],
                t2_hbm.at[pl.ds(64 * 999936, 4096)],
            )

    return k1


def _gather_kernel(num_cores):
    n_tasks = 200

    @functools.partial(
        pl.kernel,
        mesh=_mesh(),
        out_type=jax.ShapeDtypeStruct((200, 8, 32, 8, 128), jnp.float32),
        scratch_types=[
            pltpu.VMEM((200, 128), jnp.int32),
            pltpu.VMEM((2, 128, 64), jnp.float32),
            pltpu.VMEM((2, 8, 8, 128), jnp.float32),
            pltpu.SemaphoreType.DMA,
            pltpu.SemaphoreType.DMA,
        ],
        compiler_params=pltpu.CompilerParams(use_tc_tiling_on_sc=False, needs_layout_passes=False),
    )
    def k2(t2_hbm, xq_hbm, out_hbm, idxv, rows, outv, gsem, wsem):
        bt = lax.axis_index("s") * num_cores + lax.axis_index("c")
        iota = lax.broadcasted_iota(jnp.int32, (16,), 0)
        jvecs = [iota + 16 * g for g in range(8)]

        for st in range(25):
            pltpu.async_copy(
                xq_hbm.at[st, bt], idxv.at[pl.ds(8 * st, 8)], gsem
            )
        for _ in range(25):
            pltpu.make_async_copy(
                xq_hbm.at[0, 0], idxv.at[pl.ds(0, 8)], gsem
            ).wait()

        def start_gather(k, buf):
            pltpu.async_copy(t2_hbm.at[idxv.at[k]], rows.at[buf], gsem)

        def wait_gather():
            pltpu.make_async_copy(
                t2_hbm.at[pl.ds(0, 128)], rows.at[0], gsem
            ).wait()

        def wait_write():
            pltpu.make_async_copy(
                out_hbm.at[0, :, 0], outv.at[0], wsem
            ).wait()

        start_gather(0, 0)

        def task(k, buf):
            @pl.when(k + 1 < n_tasks)
            def _():
                start_gather(k + 1, 1 - buf)

            wait_gather()

            @pl.when(k >= 2)
            def _():
                wait_write()

            @plsc.parallel_loop(0, 8, unroll=2)
            def _(dt):
                for ds in range(8):
                    dsplat = jnp.full((16,), 8 * dt + ds, jnp.int32)
                    for g in range(8):
                        vals = plsc.load_gather(
                            rows.at[buf], [jvecs[g], dsplat]
                        )
                        outv[buf, dt, ds, pl.ds(16 * g, 16)] = vals

            pltpu.async_copy(outv.at[buf], out_hbm.at[k, :, bt], wsem)
            return None

        def outer(t, carry):
            task(2 * t, 0)
            task(2 * t + 1, 1)
            return carry

        lax.fori_loop(0, n_tasks // 2, outer, 0)
        wait_write()
        wait_write()

    return k2


def kernel(x, mask, table):
    del mask
    xq = x.T.reshape(25, 8, 32, 128).transpose(0, 2, 1, 3)
    tt = table.T
    info = plsc.get_sparse_core_info()
    tail = table[999936:, :].reshape(-1)
    t2 = _transpose_table_kernel(info.num_cores)(tt, tail).reshape(1000000, 64)
    out5 = _gather_kernel(info.num_cores)(t2, xq)
    return out5.transpose(2, 4, 0, 1, 3).reshape(4096, 200, 64)


# speedup vs baseline: 3.0795x; 1.5952x over previous
import functools

import jax
import jax.numpy as jnp
from jax import lax
from jax.experimental import pallas as pl
from jax.experimental.pallas import tpu as pltpu
from jax.experimental.pallas import tpu_sc as plsc


def _mesh():
    return plsc.VectorSubcoreMesh(core_axis_name="c", subcore_axis_name="s")


def _transpose_table_kernel(num_cores):
    n_main = 244

    @functools.partial(
        pl.kernel,
        mesh=_mesh(),
        out_type=jax.ShapeDtypeStruct((500000, 128), jnp.float32),
        scratch_types=[
            pltpu.VMEM((2, 64, 128), jnp.float32),
            pltpu.VMEM((2, 64, 130), jnp.float32),
            pltpu.SemaphoreType.DMA,
            pltpu.SemaphoreType.DMA,
        ],
        compiler_params=pltpu.CompilerParams(use_tc_tiling_on_sc=True, needs_layout_passes=False),
    )
    def k1(tt_hbm, tail_hbm, t2_hbm, colbuf, rowbuf, rsem, wsem):
        wid = lax.axis_index("s") * num_cores + lax.axis_index("c")
        iota = lax.broadcasted_iota(jnp.int32, (16,), 0)
        rvecs = [(iota + 16 * g) // 2 for g in range(8)]
        cbases = [((iota + 16 * g) % 2) * 64 for g in range(8)]

        def read_block(b, buf):
            pltpu.async_copy(
                tt_hbm.at[:, pl.ds(128 * b, 128)], colbuf.at[buf], rsem
            )

        def wait_read():
            pltpu.make_async_copy(
                tt_hbm.at[:, pl.ds(0, 128)], colbuf.at[0], rsem
            ).wait()

        def wait_write():
            pltpu.make_async_copy(
                t2_hbm.at[pl.ds(0, 64)],
                rowbuf.at[0, :, pl.ds(0, 128)],
                wsem,
            ).wait()

        def transpose_block(buf):
            @plsc.parallel_loop(0, 64, unroll=4)
            def _(d):
                for g in range(8):
                    vals = colbuf[buf, d, pl.ds(16 * g, 16)]
                    plsc.store_scatter(
                        rowbuf.at[buf], [rvecs[g], cbases[g] + d], vals
                    )

        def write_block(b, buf):
            pltpu.async_copy(
                rowbuf.at[buf, :, pl.ds(0, 128)],
                t2_hbm.at[pl.ds(64 * b, 64)],
                wsem,
            )

        read_block(wid, 0)

        def outer(t, carry):
            for half in range(2):
                kk = 2 * t + half
                b = wid + 32 * kk

                @pl.when(kk + 1 < n_main)
                def _():
                    read_block(wid + 32 * (kk + 1), 1 - half)

                wait_read()

                @pl.when(kk >= 2)
                def _():
                    wait_write()

                transpose_block(half)
                write_block(b, half)
            return carry

        lax.fori_loop(0, n_main // 2, outer, 0)
        wait_write()
        wait_write()

        @pl.when(wid < 4)
        def _():
            b = 7808 + wid
            pltpu.sync_copy(tt_hbm.at[:, pl.ds(128 * b, 128)], colbuf.at[0])
            transpose_block(0)
            pltpu.sync_copy(
                rowbuf.at[0, :, pl.ds(0, 128)],
                t2_hbm.at[pl.ds(64 * b, 64)],
            )

        @pl.when(wid == 4)
        def _():
            pltpu.sync_copy(tail_hbm, colbuf.at[0, pl.ds(0, 32), :])
            pltpu.sync_copy(
                colbuf.at[0, pl.ds(0, 32), :],
                t2_hbm.at[pl.ds(499968, 32)],
            )

    return k1


def _gather_kernel(num_cores):
    n_tasks = 200

    @functools.partial(
        pl.kernel,
        mesh=_mesh(),
        out_type=jax.ShapeDtypeStruct((200, 8, 32, 8, 128), jnp.float32),
        scratch_types=[
            pltpu.VMEM((200, 128), jnp.int32),
            pltpu.VMEM((2, 128, 64), jnp.float32),
            pltpu.VMEM((2, 64, 129), jnp.float32),
            pltpu.SemaphoreType.DMA,
            pltpu.SemaphoreType.DMA,
        ],
        compiler_params=pltpu.CompilerParams(use_tc_tiling_on_sc=False, needs_layout_passes=False),
    )
    def k2(t2_hbm, xq_hbm, out_hbm, idxv, rows, outv, gsem, wsem):
        bt = lax.axis_index("s") * num_cores + lax.axis_index("c")
        iota = lax.broadcasted_iota(jnp.int32, (16,), 0)
        dvecs = [iota + 16 * h for h in range(4)]

        for st in range(25):
            pltpu.async_copy(
                xq_hbm.at[st, bt], idxv.at[pl.ds(8 * st, 8)], gsem
            )
        for _ in range(25):
            pltpu.make_async_copy(
                xq_hbm.at[0, 0], idxv.at[pl.ds(0, 8)], gsem
            ).wait()

        def start_gather(k, buf):
            pltpu.async_copy(t2_hbm.at[idxv.at[k]], rows.at[buf], gsem)

        def wait_gather():
            pltpu.make_async_copy(
                t2_hbm.at[pl.ds(0, 128)], rows.at[0], gsem
            ).wait()

        def wait_write():
            pltpu.make_async_copy(
                t2_hbm.at[pl.ds(0, 128)], rows.at[0], wsem
            ).wait()

        start_gather(0, 0)

        def task(k, buf):
            @pl.when(k + 1 < n_tasks)
            def _():
                start_gather(k + 1, 1 - buf)

            wait_gather()

            @pl.when(k >= 2)
            def _():
                wait_write()

            @plsc.parallel_loop(0, 128, unroll=8)
            def _(j):
                jsplat = jnp.full((16,), j, jnp.int32)
                for h in range(4):
                    vals = rows[buf, j, pl.ds(16 * h, 16)]
                    plsc.store_scatter(
                        outv.at[buf], [dvecs[h], jsplat], vals
                    )

            for dt in range(8):
                pltpu.async_copy(
                    outv.at[buf, pl.ds(8 * dt, 8), pl.ds(0, 128)],
                    out_hbm.at[k, dt, bt],
                    wsem,
                )
            return None

        def outer(t, carry):
            task(2 * t, 0)
            task(2 * t + 1, 1)
            return carry

        lax.fori_loop(0, n_tasks // 2, outer, 0)
        wait_write()
        wait_write()

    return k2


def kernel(x, mask, table):
    del mask
    xq = x.T.reshape(25, 8, 32, 128).transpose(0, 2, 1, 3)
    tt = table.T
    info = plsc.get_sparse_core_info()
    tail = table[999936:, :].reshape(32, 128)
    t2 = _transpose_table_kernel(info.num_cores)(tt, tail).reshape(1000000, 64)
    out5 = _gather_kernel(info.num_cores)(t2, xq)
    return out5.transpose(2, 4, 0, 1, 3).reshape(4096, 200, 64)
